# baseline (speedup 1.0000x reference)
import functools
import math

import jax
import jax.numpy as jnp
from jax import lax
from jax.experimental import pallas as pl
from jax.experimental.pallas import tpu as pltpu
from jax.experimental.pallas import tpu_sc as plsc

N = 10000
E = 10000
M = 320000
D = 128
H = D // 2
NCLS = 40
NLAYERS = 4
ALPHA = 0.1
LAMDA = 0.5

NC, NS = 2, 16
NW = NC * NS
K = 100
ROWS = M // K
TROWS = ROWS // NS
NB = 4
CB = TROWS // NB
CP = 80
NCHUNK = N // CP
CITER = (NCHUNK + NS - 1) // NS

_mesh = plsc.VectorSubcoreMesh(
    core_axis_name="c", subcore_axis_name="s", num_cores=NC, num_subcores=NS
)


def _gather_scatter_body(table2, srcA, srcB, dst2d, out, src_v, dst_v,
                         rows0, rows1, rows2, rows3,
                         zero_v, acc, sem0, sem1, sem2, sem3):
    bufs = (rows0, rows1, rows2, rows3)
    sems = (sem0, sem1, sem2, sem3)
    cid = lax.axis_index("c")
    sid = lax.axis_index("s")

    base = sid * TROWS

    @pl.when(cid == 0)
    def _():
        pltpu.sync_copy(srcA.at[pl.ds(base, TROWS)], src_v)

    @pl.when(cid == 1)
    def _():
        pltpu.sync_copy(srcB.at[pl.ds(base, TROWS)], src_v)

    pltpu.sync_copy(dst2d.at[pl.ds(base, TROWS)], dst_v)

    def zfill(i, _):
        r = i // (H // 16)
        c = i % (H // 16)
        zero_v[r, pl.ds(c * 16, 16)] = jnp.zeros((16,), jnp.float32)
        return 0
    lax.fori_loop(0, CP * (H // 16), zfill, 0)

    def zcopy(k, _):
        q = k * NS + sid
        @pl.when(q < NCHUNK)
        def _():
            pltpu.sync_copy(zero_v, acc.at[pl.ds(q * CP, CP)])
        return 0
    lax.fori_loop(0, CITER, zcopy, 0)
    plsc.subcore_barrier()

    for b in range(NB):
        pltpu.async_copy(table2.at[src_v.at[b]], bufs[b], sems[b])

    def outer(g, _):
        for b in range(NB):
            j = g * NB + b
            pltpu.make_async_copy(table2.at[src_v.at[j]], bufs[b],
                                  sems[b]).wait()
            pltpu.sync_copy(bufs[b], acc.at[dst_v.at[j]], add=True)
            pltpu.async_copy(table2.at[src_v.at[j + NB]], bufs[b], sems[b])
        return 0
    lax.fori_loop(0, CB - 1, outer, 0)

    for b in range(NB):
        j = (CB - 1) * NB + b
        pltpu.make_async_copy(table2.at[src_v.at[j]], bufs[b], sems[b]).wait()
        pltpu.sync_copy(bufs[b], acc.at[dst_v.at[j]], add=True)
    plsc.subcore_barrier()

    def wback(k, _):
        q = k * NS + sid
        @pl.when(q < NCHUNK)
        def _():
            r0 = q * CP
            pltpu.sync_copy(acc.at[pl.ds(r0, CP)],
                            out.at[pl.ds(cid * N + r0, CP)])
        return 0
    lax.fori_loop(0, CITER, wback, 0)


_sc_gather_scatter = pl.kernel(
    _gather_scatter_body,
    out_type=[jax.ShapeDtypeStruct((NC * N, H), jnp.float32)],
    mesh=_mesh,
    compiler_params=pltpu.CompilerParams(use_tc_tiling_on_sc=False),
    scratch_types=[
        pltpu.VMEM((TROWS, K), jnp.int32),
        pltpu.VMEM((TROWS, K), jnp.int32),
        pltpu.VMEM((K, H), jnp.float32),
        pltpu.VMEM((K, H), jnp.float32),
        pltpu.VMEM((K, H), jnp.float32),
        pltpu.VMEM((K, H), jnp.float32),
        pltpu.VMEM((CP, H), jnp.float32),
        pltpu.VMEM_SHARED((N, H), jnp.float32),
        pltpu.SemaphoreType.DMA,
        pltpu.SemaphoreType.DMA,
        pltpu.SemaphoreType.DMA,
        pltpu.SemaphoreType.DMA,
    ],
)


def _counts_body(v2d, e2d, outv, oute, vidx, eidx, ones_v, zero_v, accv, acce):
    cid = lax.axis_index("c")
    sid = lax.axis_index("s")

    def ofill(r, _):
        ones_v[r, :] = jnp.ones((16,), jnp.float32)
        return 0
    lax.fori_loop(0, K, ofill, 0)

    def zfill(r, _):
        zero_v[r, :] = jnp.zeros((16,), jnp.float32)
        return 0
    lax.fori_loop(0, CP, zfill, 0)

    def zcopy(k, _):
        q = k * NS + sid
        @pl.when(q < NCHUNK)
        def _():
            pltpu.sync_copy(zero_v, accv.at[pl.ds(q * CP, CP)])
            pltpu.sync_copy(zero_v, acce.at[pl.ds(q * CP, CP)])
        return 0
    lax.fori_loop(0, CITER, zcopy, 0)
    plsc.subcore_barrier()

    base = sid * TROWS
    pltpu.sync_copy(v2d.at[pl.ds(base, TROWS)], vidx)
    pltpu.sync_copy(e2d.at[pl.ds(base, TROWS)], eidx)

    def step(j, _):
        pltpu.sync_copy(ones_v, accv.at[vidx.at[j]], add=True)
        pltpu.sync_copy(ones_v, acce.at[eidx.at[j]], add=True)
        return 0
    lax.fori_loop(0, TROWS, step, 0)
    plsc.subcore_barrier()

    def wback(k, _):
        q = k * NS + sid
        @pl.when(q < NCHUNK)
        def _():
            r0 = q * CP
            pltpu.sync_copy(accv.at[pl.ds(r0, CP)],
                            outv.at[pl.ds(cid * N + r0, CP)])
            pltpu.sync_copy(acce.at[pl.ds(r0, CP)],
                            oute.at[pl.ds(cid * N + r0, CP)])
        return 0
    lax.fori_loop(0, CITER, wback, 0)


_sc_counts = pl.kernel(
    _counts_body,
    out_type=[
        jax.ShapeDtypeStruct((NC * N, 16), jnp.float32),
        jax.ShapeDtypeStruct((NC * E, 16), jnp.float32),
    ],
    mesh=_mesh,
    compiler_params=pltpu.CompilerParams(use_tc_tiling_on_sc=False),
    scratch_types=[
        pltpu.VMEM((TROWS, K), jnp.int32),
        pltpu.VMEM((TROWS, K), jnp.int32),
        pltpu.VMEM((K, 16), jnp.float32),
        pltpu.VMEM((CP, 16), jnp.float32),
        pltpu.VMEM_SHARED((N, 16), jnp.float32),
        pltpu.VMEM_SHARED((E, 16), jnp.float32),
    ],
)


_BR = 1000
_GRID = N // _BR

_SPEC_L = pl.BlockSpec((1, _BR, H), lambda i: (0, i, 0))
_SPEC_R = pl.BlockSpec((1, _BR, H), lambda i: (1, i, 0))
_SPEC_LR = pl.BlockSpec((2, _BR, H), lambda i: (0, i, 0))
_SPEC_CNT = pl.BlockSpec((_BR, 16), lambda i: (i, 0))
_OUT3 = jax.ShapeDtypeStruct((2, N, H), jnp.float32)


def _lin_relu_body(x_ref, w_ref, b_ref, o_ref):
    acc = lax.dot_general(x_ref[...], w_ref[...], (((1,), (1,)), ((), ())),
                          preferred_element_type=jnp.float32)
    h = jnp.maximum(acc + b_ref[...], 0.0)
    o_ref[0] = h[:, :H]
    o_ref[1] = h[:, H:]


def _tc_linear_relu(x, W, b):
    return pl.pallas_call(
        _lin_relu_body,
        grid=(_GRID,),
        in_specs=[
            pl.BlockSpec((_BR, D), lambda i: (i, 0)),
            pl.BlockSpec((D, D), lambda i: (0, 0)),
            pl.BlockSpec((1, D), lambda i: (0, 0)),
        ],
        out_specs=_SPEC_LR,
        out_shape=_OUT3,
    )(x, W, b)


def _combine_body(p_ref, c_ref, o_ref):
    cnt = jnp.maximum(c_ref[:, 0:1], 1.0)
    o_ref[0] = p_ref[0] / cnt
    o_ref[1] = p_ref[1] / cnt


def _tc_combine(p3, c):
    return pl.pallas_call(
        _combine_body,
        grid=(_GRID,),
        in_specs=[_SPEC_LR, _SPEC_CNT],
        out_specs=_SPEC_LR,
        out_shape=_OUT3,
    )(p3, c)


def _layer_body(q_ref, c_ref, h0_ref, w_ref, o_ref, *, beta):
    cnt = jnp.maximum(c_ref[:, 0:1], 1.0)
    xv = jnp.concatenate([q_ref[0], q_ref[1]], axis=1) / cnt
    h0 = jnp.concatenate([h0_ref[0], h0_ref[1]], axis=1)
    xi = (1.0 - ALPHA) * xv + ALPHA * h0
    mm = lax.dot_general(xi, w_ref[...], (((1,), (1,)), ((), ())),
                         preferred_element_type=jnp.float32)
    h = jnp.maximum((1.0 - beta) * xi + beta * mm, 0.0)
    o_ref[0] = h[:, :H]
    o_ref[1] = h[:, H:]


def _tc_layer(q3, c, h03, W, beta):
    return pl.pallas_call(
        functools.partial(_layer_body, beta=beta),
        grid=(_GRID,),
        in_specs=[_SPEC_LR, _SPEC_CNT, _SPEC_LR,
                  pl.BlockSpec((D, D), lambda i: (0, 0))],
        out_specs=_SPEC_LR,
        out_shape=_OUT3,
    )(q3, c, h03, W)


def _out_body(h_ref, w_ref, b_ref, o_ref):
    h = jnp.concatenate([h_ref[0], h_ref[1]], axis=1)
    acc = lax.dot_general(h, w_ref[...], (((1,), (1,)), ((), ())),
                          preferred_element_type=jnp.float32)
    o_ref[...] = acc + b_ref[...]


def _tc_out(h, W, b):
    return pl.pallas_call(
        _out_body,
        grid=(_GRID,),
        in_specs=[
            _SPEC_LR,
            pl.BlockSpec((NCLS, D), lambda i: (0, 0)),
            pl.BlockSpec((1, NCLS), lambda i: (0, 0)),
        ],
        out_specs=pl.BlockSpec((_BR, NCLS), lambda i: (i, 0)),
        out_shape=jax.ShapeDtypeStruct((N, NCLS), jnp.float32),
    )(h, W, b)


def kernel(x, edge_index, W0, b0, Wconvs, Wout, bout):
    vertex2d = edge_index[0].reshape(ROWS, K)
    edges2d = edge_index[1].reshape(ROWS, K)
    vB = vertex2d + N
    eB = edges2d + N

    cntv, cnte = _sc_counts(vertex2d, edges2d)

    h3 = _tc_linear_relu(x, W0, b0.reshape(1, D))
    h03 = h3
    cnte1 = cnte[:N]
    cntv1 = cntv[:N]
    for i in range(NLAYERS):
        beta = math.log(LAMDA / (i + 1) + 1.0)
        pe = _sc_gather_scatter(h3.reshape(NC * N, H), vertex2d, vB,
                                edges2d)[0]
        xe3 = _tc_combine(pe.reshape(NC, N, H), cnte1)
        pv = _sc_gather_scatter(xe3.reshape(NC * N, H), edges2d, eB,
                                vertex2d)[0]
        h3 = _tc_layer(pv.reshape(NC, N, H), cntv1, h03, Wconvs[i], beta)

    return _tc_out(h3, Wout, bout.reshape(1, NCLS))


# speedup vs baseline: 1.0013x; 1.0013x over previous
import functools
import math

import jax
import jax.numpy as jnp
from jax import lax
from jax.experimental import pallas as pl
from jax.experimental.pallas import tpu as pltpu
from jax.experimental.pallas import tpu_sc as plsc

N = 10000
E = 10000
M = 320000
D = 128
H = D // 2
NCLS = 40
NLAYERS = 4
ALPHA = 0.1
LAMDA = 0.5

NC, NS = 2, 16
NW = NC * NS
K = 100
ROWS = M // K
TROWS = ROWS // NS
NB = 4
CB = TROWS // NB
CP = 80
NCHUNK = N // CP
CITER = (NCHUNK + NS - 1) // NS

_mesh = plsc.VectorSubcoreMesh(
    core_axis_name="c", subcore_axis_name="s", num_cores=NC, num_subcores=NS
)


def _gather_scatter_body(table2, srcA, srcB, dst2d, out, src_v, dst_v,
                         rows0, rows1, rows2, rows3,
                         zero_v, acc, sem0, sem1, sem2, sem3):
    bufs = (rows0, rows1, rows2, rows3)
    sems = (sem0, sem1, sem2, sem3)
    cid = lax.axis_index("c")
    sid = lax.axis_index("s")

    base = sid * TROWS

    @pl.when(cid == 0)
    def _():
        pltpu.sync_copy(srcA.at[pl.ds(base, TROWS)], src_v)

    @pl.when(cid == 1)
    def _():
        pltpu.sync_copy(srcB.at[pl.ds(base, TROWS)], src_v)

    pltpu.sync_copy(dst2d.at[pl.ds(base, TROWS)], dst_v)

    def zfill(i, _):
        r = i // (H // 16)
        c = i % (H // 16)
        zero_v[r, pl.ds(c * 16, 16)] = jnp.zeros((16,), jnp.float32)
        return 0
    lax.fori_loop(0, CP * (H // 16), zfill, 0)

    def zcopy(k, _):
        q = k * NS + sid
        @pl.when(q < NCHUNK)
        def _():
            pltpu.sync_copy(zero_v, acc.at[pl.ds(q * CP, CP)])
        return 0
    lax.fori_loop(0, CITER, zcopy, 0)
    plsc.subcore_barrier()

    for b in range(NB):
        pltpu.async_copy(table2.at[src_v.at[b]], bufs[b], sems[b])

    def outer(g, _):
        for b in range(NB):
            j = g * NB + b
            pltpu.make_async_copy(table2.at[src_v.at[j]], bufs[b],
                                  sems[b]).wait()
            pltpu.sync_copy(bufs[b], acc.at[dst_v.at[j]], add=True)
            pltpu.async_copy(table2.at[src_v.at[j + NB]], bufs[b], sems[b])
        return 0
    lax.fori_loop(0, CB - 1, outer, 0)

    for b in range(NB):
        j = (CB - 1) * NB + b
        pltpu.make_async_copy(table2.at[src_v.at[j]], bufs[b], sems[b]).wait()
        pltpu.sync_copy(bufs[b], acc.at[dst_v.at[j]], add=True)
    plsc.subcore_barrier()

    def wback(k, _):
        q = k * NS + sid
        @pl.when(q < NCHUNK)
        def _():
            r0 = q * CP
            pltpu.sync_copy(acc.at[pl.ds(r0, CP)],
                            out.at[pl.ds(cid * N + r0, CP)])
        return 0
    lax.fori_loop(0, CITER, wback, 0)


_sc_gather_scatter = pl.kernel(
    _gather_scatter_body,
    out_type=[jax.ShapeDtypeStruct((NC * N, H), jnp.float32)],
    mesh=_mesh,
    compiler_params=pltpu.CompilerParams(use_tc_tiling_on_sc=False),
    scratch_types=[
        pltpu.VMEM((TROWS, K), jnp.int32),
        pltpu.VMEM((TROWS, K), jnp.int32),
        pltpu.VMEM((K, H), jnp.float32),
        pltpu.VMEM((K, H), jnp.float32),
        pltpu.VMEM((K, H), jnp.float32),
        pltpu.VMEM((K, H), jnp.float32),
        pltpu.VMEM((CP, H), jnp.float32),
        pltpu.VMEM_SHARED((N, H), jnp.float32),
        pltpu.SemaphoreType.DMA,
        pltpu.SemaphoreType.DMA,
        pltpu.SemaphoreType.DMA,
        pltpu.SemaphoreType.DMA,
    ],
)


def _counts_body(v2d, e2d, outv, oute, vidx, eidx, ones_v, zero_v, accv, acce):
    cid = lax.axis_index("c")
    sid = lax.axis_index("s")

    def ofill(r, _):
        ones_v[r, :] = jnp.ones((16,), jnp.float32)
        return 0
    lax.fori_loop(0, K, ofill, 0)

    def zfill(r, _):
        zero_v[r, :] = jnp.zeros((16,), jnp.float32)
        return 0
    lax.fori_loop(0, CP, zfill, 0)

    def zcopy(k, _):
        q = k * NS + sid
        @pl.when(q < NCHUNK)
        def _():
            pltpu.sync_copy(zero_v, accv.at[pl.ds(q * CP, CP)])
            pltpu.sync_copy(zero_v, acce.at[pl.ds(q * CP, CP)])
        return 0
    lax.fori_loop(0, CITER, zcopy, 0)
    plsc.subcore_barrier()

    base = sid * TROWS
    pltpu.sync_copy(v2d.at[pl.ds(base, TROWS)], vidx)
    pltpu.sync_copy(e2d.at[pl.ds(base, TROWS)], eidx)

    def step(j, _):
        pltpu.sync_copy(ones_v, accv.at[vidx.at[j]], add=True)
        pltpu.sync_copy(ones_v, acce.at[eidx.at[j]], add=True)
        return 0
    lax.fori_loop(0, TROWS, step, 0)
    plsc.subcore_barrier()

    def wback(k, _):
        q = k * NS + sid
        @pl.when(q < NCHUNK)
        def _():
            r0 = q * CP
            pltpu.sync_copy(accv.at[pl.ds(r0, CP)],
                            outv.at[pl.ds(cid * N + r0, CP)])
            pltpu.sync_copy(acce.at[pl.ds(r0, CP)],
                            oute.at[pl.ds(cid * N + r0, CP)])
        return 0
    lax.fori_loop(0, CITER, wback, 0)


_sc_counts = pl.kernel(
    _counts_body,
    out_type=[
        jax.ShapeDtypeStruct((NC * N, 16), jnp.float32),
        jax.ShapeDtypeStruct((NC * E, 16), jnp.float32),
    ],
    mesh=_mesh,
    compiler_params=pltpu.CompilerParams(use_tc_tiling_on_sc=False),
    scratch_types=[
        pltpu.VMEM((TROWS, K), jnp.int32),
        pltpu.VMEM((TROWS, K), jnp.int32),
        pltpu.VMEM((K, 16), jnp.float32),
        pltpu.VMEM((CP, 16), jnp.float32),
        pltpu.VMEM_SHARED((N, 16), jnp.float32),
        pltpu.VMEM_SHARED((E, 16), jnp.float32),
    ],
)


_BR = 1000
_GRID = N // _BR

_SPEC_L = pl.BlockSpec((_BR, H), lambda i: (i, 0))
_SPEC_R = pl.BlockSpec((_BR, H), lambda i: (i + _GRID, 0))
_SPEC_BI = pl.BlockSpec((2 * _BR, H), lambda i: (i, 0))
_SPEC_CNT = pl.BlockSpec((_BR, 16), lambda i: (i, 0))
_OUT2 = jax.ShapeDtypeStruct((NC * N, H), jnp.float32)


def _lin_relu_body(x_ref, w_ref, b_ref, o_ref):
    acc = lax.dot_general(x_ref[...], w_ref[...], (((1,), (1,)), ((), ())),
                          preferred_element_type=jnp.float32)
    h = jnp.maximum(acc + b_ref[...], 0.0)
    o_ref[:_BR] = h[:, :H]
    o_ref[_BR:] = h[:, H:]


def _tc_linear_relu(x, W, b):
    return pl.pallas_call(
        _lin_relu_body,
        grid=(_GRID,),
        in_specs=[
            pl.BlockSpec((_BR, D), lambda i: (i, 0)),
            pl.BlockSpec((D, D), lambda i: (0, 0)),
            pl.BlockSpec((1, D), lambda i: (0, 0)),
        ],
        out_specs=_SPEC_BI,
        out_shape=_OUT2,
    )(x, W, b)


def _combine_body(pl_ref, pr_ref, c_ref, o_ref):
    cnt = jnp.maximum(c_ref[:, 0:1], 1.0)
    o_ref[:_BR] = pl_ref[...] / cnt
    o_ref[_BR:] = pr_ref[...] / cnt


def _tc_combine(p, c):
    return pl.pallas_call(
        _combine_body,
        grid=(_GRID,),
        in_specs=[_SPEC_L, _SPEC_R, _SPEC_CNT],
        out_specs=_SPEC_BI,
        out_shape=_OUT2,
    )(p, p, c)


def _layer_body(ql_ref, qr_ref, c_ref, h0_ref, w_ref, o_ref, *, beta):
    cnt = jnp.maximum(c_ref[:, 0:1], 1.0)
    xv = jnp.concatenate([ql_ref[...], qr_ref[...]], axis=1) / cnt
    h0 = jnp.concatenate([h0_ref[:_BR], h0_ref[_BR:]], axis=1)
    xi = (1.0 - ALPHA) * xv + ALPHA * h0
    mm = lax.dot_general(xi, w_ref[...], (((1,), (1,)), ((), ())),
                         preferred_element_type=jnp.float32)
    h = jnp.maximum((1.0 - beta) * xi + beta * mm, 0.0)
    o_ref[:_BR] = h[:, :H]
    o_ref[_BR:] = h[:, H:]


def _tc_layer(q, c, h0, W, beta):
    return pl.pallas_call(
        functools.partial(_layer_body, beta=beta),
        grid=(_GRID,),
        in_specs=[_SPEC_L, _SPEC_R, _SPEC_CNT, _SPEC_BI,
                  pl.BlockSpec((D, D), lambda i: (0, 0))],
        out_specs=_SPEC_BI,
        out_shape=_OUT2,
    )(q, q, c, h0, W)


def _out_body(h_ref, w_ref, b_ref, o_ref):
    h = jnp.concatenate([h_ref[:_BR], h_ref[_BR:]], axis=1)
    acc = lax.dot_general(h, w_ref[...], (((1,), (1,)), ((), ())),
                          preferred_element_type=jnp.float32)
    o_ref[...] = acc + b_ref[...]


def _tc_out(h, W, b):
    return pl.pallas_call(
        _out_body,
        grid=(_GRID,),
        in_specs=[
            _SPEC_BI,
            pl.BlockSpec((NCLS, D), lambda i: (0, 0)),
            pl.BlockSpec((1, NCLS), lambda i: (0, 0)),
        ],
        out_specs=pl.BlockSpec((_BR, NCLS), lambda i: (i, 0)),
        out_shape=jax.ShapeDtypeStruct((N, NCLS), jnp.float32),
    )(h, W, b)


def kernel(x, edge_index, W0, b0, Wconvs, Wout, bout):
    vertex2d = edge_index[0].reshape(ROWS, K)
    edges2d = edge_index[1].reshape(ROWS, K)
    vT = (vertex2d // _BR) * (2 * _BR) + (vertex2d % _BR)
    vTB = vT + _BR
    eT = (edges2d // _BR) * (2 * _BR) + (edges2d % _BR)
    eTB = eT + _BR

    cntv, cnte = _sc_counts(vertex2d, edges2d)

    h = _tc_linear_relu(x, W0, b0.reshape(1, D))
    h0 = h
    cnte1 = cnte[:N]
    cntv1 = cntv[:N]
    for i in range(NLAYERS):
        beta = math.log(LAMDA / (i + 1) + 1.0)
        pe = _sc_gather_scatter(h, vT, vTB, edges2d)[0]
        xe = _tc_combine(pe, cnte1)
        pv = _sc_gather_scatter(xe, eT, eTB, vertex2d)[0]
        h = _tc_layer(pv, cntv1, h0, Wconvs[i], beta)

    return _tc_out(h, Wout, bout.reshape(1, NCLS))


# speedup vs baseline: 1.0342x; 1.0328x over previous
import functools
import math

import jax
import jax.numpy as jnp
from jax import lax
from jax.experimental import pallas as pl
from jax.experimental.pallas import tpu as pltpu
from jax.experimental.pallas import tpu_sc as plsc

N = 10000
E = 10000
M = 320000
D = 128
H = D // 2
NCLS = 40
NLAYERS = 4
ALPHA = 0.1
LAMDA = 0.5

NC, NS = 2, 16
NW = NC * NS
K = 100
ROWS = M // K
TROWS = ROWS // NS
NB = 4
CB = TROWS // NB
CP = 80
NCHUNK = N // CP
CITER = (NCHUNK + NS - 1) // NS

_mesh = plsc.VectorSubcoreMesh(
    core_axis_name="c", subcore_axis_name="s", num_cores=NC, num_subcores=NS
)


def _gather_scatter_body(table2, srcA, srcB, dst2d, out, src_v, dst_v,
                         rows0, rows1, rows2, rows3,
                         zero_v, acc, sem0, sem1, sem2, sem3):
    bufs = (rows0, rows1, rows2, rows3)
    sems = (sem0, sem1, sem2, sem3)
    cid = lax.axis_index("c")
    sid = lax.axis_index("s")

    base = sid * TROWS

    @pl.when(cid == 0)
    def _():
        pltpu.sync_copy(srcA.at[pl.ds(base, TROWS)], src_v)

    @pl.when(cid == 1)
    def _():
        pltpu.sync_copy(srcB.at[pl.ds(base, TROWS)], src_v)

    pltpu.sync_copy(dst2d.at[pl.ds(base, TROWS)], dst_v)

    def zfill(i, _):
        r = i // (H // 16)
        c = i % (H // 16)
        zero_v[r, pl.ds(c * 16, 16)] = jnp.zeros((16,), jnp.float32)
        return 0
    lax.fori_loop(0, CP * (H // 16), zfill, 0)

    def zcopy(k, _):
        q = k * NS + sid
        @pl.when(q < NCHUNK)
        def _():
            pltpu.sync_copy(zero_v, acc.at[pl.ds(q * CP, CP)])
        return 0
    lax.fori_loop(0, CITER, zcopy, 0)
    plsc.subcore_barrier()

    for b in range(NB):
        pltpu.async_copy(table2.at[src_v.at[b]], bufs[b], sems[b])

    def outer(g, _):
        for b in range(NB):
            j = g * NB + b
            pltpu.make_async_copy(table2.at[src_v.at[j]], bufs[b],
                                  sems[b]).wait()
            pltpu.sync_copy(bufs[b], acc.at[dst_v.at[j]], add=True)
            pltpu.async_copy(table2.at[src_v.at[j + NB]], bufs[b], sems[b])
        return 0
    lax.fori_loop(0, CB - 1, outer, 0)

    for b in range(NB):
        j = (CB - 1) * NB + b
        pltpu.make_async_copy(table2.at[src_v.at[j]], bufs[b], sems[b]).wait()
        pltpu.sync_copy(bufs[b], acc.at[dst_v.at[j]], add=True)
    plsc.subcore_barrier()

    def wback(k, _):
        q = k * NS + sid
        @pl.when(q < NCHUNK)
        def _():
            r0 = q * CP
            pltpu.sync_copy(acc.at[pl.ds(r0, CP)],
                            out.at[pl.ds(cid * N + r0, CP)])
        return 0
    lax.fori_loop(0, CITER, wback, 0)


def _gather_scatter_scaled_body(table2, srcA, srcB, dst2d, inv16, out,
                                src_v, dst_v, rows0, rows1, rows2, rows3,
                                zero_v, tmp_v, inv_v, acc,
                                sem0, sem1, sem2, sem3):
    bufs = (rows0, rows1, rows2, rows3)
    sems = (sem0, sem1, sem2, sem3)
    cid = lax.axis_index("c")
    sid = lax.axis_index("s")

    base = sid * TROWS

    @pl.when(cid == 0)
    def _():
        pltpu.sync_copy(srcA.at[pl.ds(base, TROWS)], src_v)

    @pl.when(cid == 1)
    def _():
        pltpu.sync_copy(srcB.at[pl.ds(base, TROWS)], src_v)

    pltpu.sync_copy(dst2d.at[pl.ds(base, TROWS)], dst_v)

    def zfill(i, _):
        r = i // (H // 16)
        c = i % (H // 16)
        zero_v[r, pl.ds(c * 16, 16)] = jnp.zeros((16,), jnp.float32)
        return 0
    lax.fori_loop(0, CP * (H // 16), zfill, 0)

    def zcopy(k, _):
        q = k * NS + sid
        @pl.when(q < NCHUNK)
        def _():
            pltpu.sync_copy(zero_v, acc.at[pl.ds(q * CP, CP)])
        return 0
    lax.fori_loop(0, CITER, zcopy, 0)
    plsc.subcore_barrier()

    for b in range(NB):
        pltpu.async_copy(table2.at[src_v.at[b]], bufs[b], sems[b])

    def outer(g, _):
        for b in range(NB):
            j = g * NB + b
            pltpu.make_async_copy(table2.at[src_v.at[j]], bufs[b],
                                  sems[b]).wait()
            pltpu.sync_copy(bufs[b], acc.at[dst_v.at[j]], add=True)
            pltpu.async_copy(table2.at[src_v.at[j + NB]], bufs[b], sems[b])
        return 0
    lax.fori_loop(0, CB - 1, outer, 0)

    for b in range(NB):
        j = (CB - 1) * NB + b
        pltpu.make_async_copy(table2.at[src_v.at[j]], bufs[b], sems[b]).wait()
        pltpu.sync_copy(bufs[b], acc.at[dst_v.at[j]], add=True)
    plsc.subcore_barrier()

    def wback(k, _):
        q = k * NS + sid
        @pl.when(q < NCHUNK)
        def _():
            r0 = q * CP
            pltpu.sync_copy(acc.at[pl.ds(r0, CP)], tmp_v)
            pltpu.sync_copy(inv16.at[pl.ds(r0, CP)], inv_v)
            def scale(i, _):
                r = i // (H // 16)
                c = i % (H // 16)
                tmp_v[r, pl.ds(c * 16, 16)] = (
                    tmp_v[r, pl.ds(c * 16, 16)] * inv_v[r, :])
                return 0
            lax.fori_loop(0, CP * (H // 16), scale, 0)
            pltpu.sync_copy(tmp_v, out.at[pl.ds(cid * N + r0, CP)])
        return 0
    lax.fori_loop(0, CITER, wback, 0)


_sc_gather_scatter_scaled = pl.kernel(
    _gather_scatter_scaled_body,
    out_type=[jax.ShapeDtypeStruct((NC * N, H), jnp.float32)],
    mesh=_mesh,
    compiler_params=pltpu.CompilerParams(use_tc_tiling_on_sc=False),
    scratch_types=[
        pltpu.VMEM((TROWS, K), jnp.int32),
        pltpu.VMEM((TROWS, K), jnp.int32),
        pltpu.VMEM((K, H), jnp.float32),
        pltpu.VMEM((K, H), jnp.float32),
        pltpu.VMEM((K, H), jnp.float32),
        pltpu.VMEM((K, H), jnp.float32),
        pltpu.VMEM((CP, H), jnp.float32),
        pltpu.VMEM((CP, H), jnp.float32),
        pltpu.VMEM((CP, 16), jnp.float32),
        pltpu.VMEM_SHARED((N, H), jnp.float32),
        pltpu.SemaphoreType.DMA,
        pltpu.SemaphoreType.DMA,
        pltpu.SemaphoreType.DMA,
        pltpu.SemaphoreType.DMA,
    ],
)


_sc_gather_scatter = pl.kernel(
    _gather_scatter_body,
    out_type=[jax.ShapeDtypeStruct((NC * N, H), jnp.float32)],
    mesh=_mesh,
    compiler_params=pltpu.CompilerParams(use_tc_tiling_on_sc=False),
    scratch_types=[
        pltpu.VMEM((TROWS, K), jnp.int32),
        pltpu.VMEM((TROWS, K), jnp.int32),
        pltpu.VMEM((K, H), jnp.float32),
        pltpu.VMEM((K, H), jnp.float32),
        pltpu.VMEM((K, H), jnp.float32),
        pltpu.VMEM((K, H), jnp.float32),
        pltpu.VMEM((CP, H), jnp.float32),
        pltpu.VMEM_SHARED((N, H), jnp.float32),
        pltpu.SemaphoreType.DMA,
        pltpu.SemaphoreType.DMA,
        pltpu.SemaphoreType.DMA,
        pltpu.SemaphoreType.DMA,
    ],
)


def _counts_body(v2d, e2d, outv, oute, vidx, eidx, ones_v, zero_v, accv, acce):
    cid = lax.axis_index("c")
    sid = lax.axis_index("s")

    def ofill(r, _):
        ones_v[r, :] = jnp.ones((16,), jnp.float32)
        return 0
    lax.fori_loop(0, K, ofill, 0)

    def zfill(r, _):
        zero_v[r, :] = jnp.zeros((16,), jnp.float32)
        return 0
    lax.fori_loop(0, CP, zfill, 0)

    def zcopy(k, _):
        q = k * NS + sid
        @pl.when(q < NCHUNK)
        def _():
            pltpu.sync_copy(zero_v, accv.at[pl.ds(q * CP, CP)])
            pltpu.sync_copy(zero_v, acce.at[pl.ds(q * CP, CP)])
        return 0
    lax.fori_loop(0, CITER, zcopy, 0)
    plsc.subcore_barrier()

    base = sid * TROWS
    pltpu.sync_copy(v2d.at[pl.ds(base, TROWS)], vidx)
    pltpu.sync_copy(e2d.at[pl.ds(base, TROWS)], eidx)

    def step(j, _):
        pltpu.sync_copy(ones_v, accv.at[vidx.at[j]], add=True)
        pltpu.sync_copy(ones_v, acce.at[eidx.at[j]], add=True)
        return 0
    lax.fori_loop(0, TROWS, step, 0)
    plsc.subcore_barrier()

    def wback(k, _):
        q = k * NS + sid
        @pl.when(q < NCHUNK)
        def _():
            r0 = q * CP
            pltpu.sync_copy(accv.at[pl.ds(r0, CP)], zero_v)
            def invv(r, _):
                zero_v[r, :] = 1.0 / jnp.maximum(zero_v[r, :], 1.0)
                return 0
            lax.fori_loop(0, CP, invv, 0)
            pltpu.sync_copy(zero_v, outv.at[pl.ds(cid * N + r0, CP)])
            pltpu.sync_copy(acce.at[pl.ds(r0, CP)], zero_v)
            lax.fori_loop(0, CP, invv, 0)
            pltpu.sync_copy(zero_v, oute.at[pl.ds(cid * N + r0, CP)])
        return 0
    lax.fori_loop(0, CITER, wback, 0)


_sc_counts = pl.kernel(
    _counts_body,
    out_type=[
        jax.ShapeDtypeStruct((NC * N, 16), jnp.float32),
        jax.ShapeDtypeStruct((NC * E, 16), jnp.float32),
    ],
    mesh=_mesh,
    compiler_params=pltpu.CompilerParams(use_tc_tiling_on_sc=False),
    scratch_types=[
        pltpu.VMEM((TROWS, K), jnp.int32),
        pltpu.VMEM((TROWS, K), jnp.int32),
        pltpu.VMEM((K, 16), jnp.float32),
        pltpu.VMEM((CP, 16), jnp.float32),
        pltpu.VMEM_SHARED((N, 16), jnp.float32),
        pltpu.VMEM_SHARED((E, 16), jnp.float32),
    ],
)


_BR = 1000
_GRID = N // _BR

_SPEC_L = pl.BlockSpec((_BR, H), lambda i: (i, 0))
_SPEC_R = pl.BlockSpec((_BR, H), lambda i: (i + _GRID, 0))
_SPEC_BI = pl.BlockSpec((2 * _BR, H), lambda i: (i, 0))
_SPEC_CNT = pl.BlockSpec((_BR, 16), lambda i: (i, 0))
_OUT2 = jax.ShapeDtypeStruct((NC * N, H), jnp.float32)


def _lin_relu_body(x_ref, w_ref, b_ref, o_ref):
    acc = lax.dot_general(x_ref[...], w_ref[...], (((1,), (1,)), ((), ())),
                          preferred_element_type=jnp.float32)
    h = jnp.maximum(acc + b_ref[...], 0.0)
    o_ref[:_BR] = h[:, :H]
    o_ref[_BR:] = h[:, H:]


def _tc_linear_relu(x, W, b):
    return pl.pallas_call(
        _lin_relu_body,
        grid=(_GRID,),
        in_specs=[
            pl.BlockSpec((_BR, D), lambda i: (i, 0)),
            pl.BlockSpec((D, D), lambda i: (0, 0)),
            pl.BlockSpec((1, D), lambda i: (0, 0)),
        ],
        out_specs=_SPEC_BI,
        out_shape=_OUT2,
    )(x, W, b)


def _combine_body(pl_ref, pr_ref, c_ref, o_ref):
    cnt = jnp.maximum(c_ref[:, 0:1], 1.0)
    o_ref[:_BR] = pl_ref[...] / cnt
    o_ref[_BR:] = pr_ref[...] / cnt


def _tc_combine(p, c):
    return pl.pallas_call(
        _combine_body,
        grid=(_GRID,),
        in_specs=[_SPEC_L, _SPEC_R, _SPEC_CNT],
        out_specs=_SPEC_BI,
        out_shape=_OUT2,
    )(p, p, c)


def _layer_body(ql_ref, qr_ref, c_ref, h0_ref, w_ref, o_ref, *, beta):
    xv = jnp.concatenate([ql_ref[...], qr_ref[...]], axis=1) * c_ref[:, 0:1]
    h0 = jnp.concatenate([h0_ref[:_BR], h0_ref[_BR:]], axis=1)
    xi = (1.0 - ALPHA) * xv + ALPHA * h0
    mm = lax.dot_general(xi, w_ref[...], (((1,), (1,)), ((), ())),
                         preferred_element_type=jnp.float32)
    h = jnp.maximum((1.0 - beta) * xi + beta * mm, 0.0)
    o_ref[:_BR] = h[:, :H]
    o_ref[_BR:] = h[:, H:]


def _tc_layer(q, c, h0, W, beta):
    return pl.pallas_call(
        functools.partial(_layer_body, beta=beta),
        grid=(_GRID,),
        in_specs=[_SPEC_L, _SPEC_R, _SPEC_CNT, _SPEC_BI,
                  pl.BlockSpec((D, D), lambda i: (0, 0))],
        out_specs=_SPEC_BI,
        out_shape=_OUT2,
    )(q, q, c, h0, W)


def _out_body(h_ref, w_ref, b_ref, o_ref):
    h = jnp.concatenate([h_ref[:_BR], h_ref[_BR:]], axis=1)
    acc = lax.dot_general(h, w_ref[...], (((1,), (1,)), ((), ())),
                          preferred_element_type=jnp.float32)
    o_ref[...] = acc + b_ref[...]


def _tc_out(h, W, b):
    return pl.pallas_call(
        _out_body,
        grid=(_GRID,),
        in_specs=[
            _SPEC_BI,
            pl.BlockSpec((NCLS, D), lambda i: (0, 0)),
            pl.BlockSpec((1, NCLS), lambda i: (0, 0)),
        ],
        out_specs=pl.BlockSpec((_BR, NCLS), lambda i: (i, 0)),
        out_shape=jax.ShapeDtypeStruct((N, NCLS), jnp.float32),
    )(h, W, b)


def kernel(x, edge_index, W0, b0, Wconvs, Wout, bout):
    vertex2d = edge_index[0].reshape(ROWS, K)
    edges2d = edge_index[1].reshape(ROWS, K)
    vT = (vertex2d // _BR) * (2 * _BR) + (vertex2d % _BR)
    vTB = vT + _BR
    eB = edges2d + N

    invv, inve = _sc_counts(vertex2d, edges2d)

    h = _tc_linear_relu(x, W0, b0.reshape(1, D))
    h0 = h
    invv1 = invv[:N]
    for i in range(NLAYERS):
        beta = math.log(LAMDA / (i + 1) + 1.0)
        xe = _sc_gather_scatter_scaled(h, vT, vTB, edges2d, inve)[0]
        pv = _sc_gather_scatter(xe, edges2d, eB, vertex2d)[0]
        h = _tc_layer(pv, invv1, h0, Wconvs[i], beta)

    return _tc_out(h, Wout, bout.reshape(1, NCLS))


# speedup vs baseline: 1.0523x; 1.0175x over previous
import functools
import math

import jax
import jax.numpy as jnp
from jax import lax
from jax.experimental import pallas as pl
from jax.experimental.pallas import tpu as pltpu
from jax.experimental.pallas import tpu_sc as plsc

N = 10000
E = 10000
M = 320000
D = 128
H = D // 2
NCLS = 40
NLAYERS = 4
ALPHA = 0.1
LAMDA = 0.5

NC, NS = 2, 16
NW = NC * NS
K = 100
ROWS = M // K
TROWS = ROWS // NS
NB = 4
CB = TROWS // NB
CP = 80
NCHUNK = N // CP
CITER = (NCHUNK + NS - 1) // NS

_mesh = plsc.VectorSubcoreMesh(
    core_axis_name="c", subcore_axis_name="s", num_cores=NC, num_subcores=NS
)


def _gather_scatter_body(table2, srcA, srcB, dst2d, out, src_v, dst_v,
                         rows0, rows1, rows2, rows3,
                         zero_v, acc, sem0, sem1, sem2, sem3):
    bufs = (rows0, rows1, rows2, rows3)
    sems = (sem0, sem1, sem2, sem3)
    cid = lax.axis_index("c")
    sid = lax.axis_index("s")

    base = sid * TROWS

    @pl.when(cid == 0)
    def _():
        pltpu.sync_copy(srcA.at[pl.ds(base, TROWS)], src_v)

    @pl.when(cid == 1)
    def _():
        pltpu.sync_copy(srcB.at[pl.ds(base, TROWS)], src_v)

    pltpu.sync_copy(dst2d.at[pl.ds(base, TROWS)], dst_v)

    def zfill(i, _):
        r = i // (H // 16)
        c = i % (H // 16)
        zero_v[r, pl.ds(c * 16, 16)] = jnp.zeros((16,), jnp.float32)
        return 0
    lax.fori_loop(0, CP * (H // 16), zfill, 0)

    def zcopy(k, _):
        q = k * NS + sid
        @pl.when(q < NCHUNK)
        def _():
            pltpu.sync_copy(zero_v, acc.at[pl.ds(q * CP, CP)])
        return 0
    lax.fori_loop(0, CITER, zcopy, 0)
    plsc.subcore_barrier()

    for b in range(NB):
        pltpu.async_copy(table2.at[src_v.at[b]], bufs[b], sems[b])

    def outer(g, _):
        for b in range(NB):
            j = g * NB + b
            pltpu.make_async_copy(table2.at[src_v.at[j]], bufs[b],
                                  sems[b]).wait()
            pltpu.sync_copy(bufs[b], acc.at[dst_v.at[j]], add=True)
            pltpu.async_copy(table2.at[src_v.at[j + NB]], bufs[b], sems[b])
        return 0
    lax.fori_loop(0, CB - 1, outer, 0)

    for b in range(NB):
        j = (CB - 1) * NB + b
        pltpu.make_async_copy(table2.at[src_v.at[j]], bufs[b], sems[b]).wait()
        pltpu.sync_copy(bufs[b], acc.at[dst_v.at[j]], add=True)
    plsc.subcore_barrier()

    def wback(k, _):
        q = k * NS + sid
        @pl.when(q < NCHUNK)
        def _():
            r0 = q * CP
            pltpu.sync_copy(acc.at[pl.ds(r0, CP)],
                            out.at[pl.ds(cid * N + r0, CP)])
        return 0
    lax.fori_loop(0, CITER, wback, 0)


_sc_gather_scatter = pl.kernel(
    _gather_scatter_body,
    out_type=[jax.ShapeDtypeStruct((NC * N, H), jnp.float32)],
    mesh=_mesh,
    compiler_params=pltpu.CompilerParams(use_tc_tiling_on_sc=False),
    scratch_types=[
        pltpu.VMEM((TROWS, K), jnp.int32),
        pltpu.VMEM((TROWS, K), jnp.int32),
        pltpu.VMEM((K, H), jnp.float32),
        pltpu.VMEM((K, H), jnp.float32),
        pltpu.VMEM((K, H), jnp.float32),
        pltpu.VMEM((K, H), jnp.float32),
        pltpu.VMEM((CP, H), jnp.float32),
        pltpu.VMEM_SHARED((N, H), jnp.float32),
        pltpu.SemaphoreType.DMA,
        pltpu.SemaphoreType.DMA,
        pltpu.SemaphoreType.DMA,
        pltpu.SemaphoreType.DMA,
    ],
)


def _counts_body(v2d, e2d, outv, oute, vidx, eidx, ones_v, zero_v, accv, acce):
    cid = lax.axis_index("c")
    sid = lax.axis_index("s")

    def ofill(r, _):
        ones_v[r, :] = jnp.ones((16,), jnp.float32)
        return 0
    lax.fori_loop(0, K, ofill, 0)

    def zfill(r, _):
        zero_v[r, :] = jnp.zeros((16,), jnp.float32)
        return 0
    lax.fori_loop(0, CP, zfill, 0)

    def zcopy(k, _):
        q = k * NS + sid
        @pl.when(q < NCHUNK)
        def _():
            pltpu.sync_copy(zero_v, accv.at[pl.ds(q * CP, CP)])
            pltpu.sync_copy(zero_v, acce.at[pl.ds(q * CP, CP)])
        return 0
    lax.fori_loop(0, CITER, zcopy, 0)
    plsc.subcore_barrier()

    base = sid * TROWS
    pltpu.sync_copy(v2d.at[pl.ds(base, TROWS)], vidx)
    pltpu.sync_copy(e2d.at[pl.ds(base, TROWS)], eidx)

    def step(j, _):
        pltpu.sync_copy(ones_v, accv.at[vidx.at[j]], add=True)
        pltpu.sync_copy(ones_v, acce.at[eidx.at[j]], add=True)
        return 0
    lax.fori_loop(0, TROWS, step, 0)
    plsc.subcore_barrier()

    def wback(k, _):
        q = k * NS + sid
        @pl.when(q < NCHUNK)
        def _():
            r0 = q * CP
            pltpu.sync_copy(accv.at[pl.ds(r0, CP)], zero_v)
            def invv(r, _):
                zero_v[r, :] = 1.0 / jnp.maximum(zero_v[r, :], 1.0)
                return 0
            lax.fori_loop(0, CP, invv, 0)
            pltpu.sync_copy(zero_v, outv.at[pl.ds(cid * N + r0, CP)])
            pltpu.sync_copy(acce.at[pl.ds(r0, CP)], zero_v)
            lax.fori_loop(0, CP, invv, 0)
            pltpu.sync_copy(zero_v, oute.at[pl.ds(cid * N + r0, CP)])
        return 0
    lax.fori_loop(0, CITER, wback, 0)


_sc_counts = pl.kernel(
    _counts_body,
    out_type=[
        jax.ShapeDtypeStruct((NC * N, 16), jnp.float32),
        jax.ShapeDtypeStruct((NC * E, 16), jnp.float32),
    ],
    mesh=_mesh,
    compiler_params=pltpu.CompilerParams(use_tc_tiling_on_sc=False),
    scratch_types=[
        pltpu.VMEM((TROWS, K), jnp.int32),
        pltpu.VMEM((TROWS, K), jnp.int32),
        pltpu.VMEM((K, 16), jnp.float32),
        pltpu.VMEM((CP, 16), jnp.float32),
        pltpu.VMEM_SHARED((N, 16), jnp.float32),
        pltpu.VMEM_SHARED((E, 16), jnp.float32),
    ],
)


_BR = 1000
_GRID = N // _BR

_SPEC_L = pl.BlockSpec((_BR, H), lambda i: (i, 0))
_SPEC_R = pl.BlockSpec((_BR, H), lambda i: (i + _GRID, 0))
_SPEC_CNT = pl.BlockSpec((_BR, 16), lambda i: (i, 0))
_SPEC_D = pl.BlockSpec((_BR, D), lambda i: (i, 0))
_OUTD = jax.ShapeDtypeStruct((N, D), jnp.float32)


def _lin_relu_body(x_ref, w_ref, b_ref, o_ref):
    acc = lax.dot_general(x_ref[...], w_ref[...], (((1,), (1,)), ((), ())),
                          preferred_element_type=jnp.float32)
    o_ref[...] = jnp.maximum(acc + b_ref[...], 0.0)


def _tc_linear_relu(x, W, b):
    return pl.pallas_call(
        _lin_relu_body,
        grid=(_GRID,),
        in_specs=[
            pl.BlockSpec((_BR, D), lambda i: (i, 0)),
            pl.BlockSpec((D, D), lambda i: (0, 0)),
            pl.BlockSpec((1, D), lambda i: (0, 0)),
        ],
        out_specs=_SPEC_D,
        out_shape=_OUTD,
    )(x, W, b)


def _combine_body(pl_ref, pr_ref, c_ref, o_ref):
    s = jnp.concatenate([pl_ref[...], pr_ref[...]], axis=1)
    o_ref[...] = s * c_ref[:, 0:1]


def _tc_combine(p, c):
    return pl.pallas_call(
        _combine_body,
        grid=(_GRID,),
        in_specs=[_SPEC_L, _SPEC_R, _SPEC_CNT],
        out_specs=_SPEC_D,
        out_shape=_OUTD,
    )(p, p, c)


def _layer_body(ql_ref, qr_ref, c_ref, h0_ref, w_ref, o_ref, *, beta):
    xv = jnp.concatenate([ql_ref[...], qr_ref[...]], axis=1) * c_ref[:, 0:1]
    xi = (1.0 - ALPHA) * xv + ALPHA * h0_ref[...]
    mm = lax.dot_general(xi, w_ref[...], (((1,), (1,)), ((), ())),
                         preferred_element_type=jnp.float32)
    o_ref[...] = jnp.maximum((1.0 - beta) * xi + beta * mm, 0.0)


def _tc_layer(q, c, h0, W, beta):
    return pl.pallas_call(
        functools.partial(_layer_body, beta=beta),
        grid=(_GRID,),
        in_specs=[_SPEC_L, _SPEC_R, _SPEC_CNT, _SPEC_D,
                  pl.BlockSpec((D, D), lambda i: (0, 0))],
        out_specs=_SPEC_D,
        out_shape=_OUTD,
    )(q, q, c, h0, W)


def _out_body(h_ref, w_ref, b_ref, o_ref):
    acc = lax.dot_general(h_ref[...], w_ref[...], (((1,), (1,)), ((), ())),
                          preferred_element_type=jnp.float32)
    o_ref[...] = acc + b_ref[...]


def _tc_out(h, W, b):
    return pl.pallas_call(
        _out_body,
        grid=(_GRID,),
        in_specs=[
            _SPEC_D,
            pl.BlockSpec((NCLS, D), lambda i: (0, 0)),
            pl.BlockSpec((1, NCLS), lambda i: (0, 0)),
        ],
        out_specs=pl.BlockSpec((_BR, NCLS), lambda i: (i, 0)),
        out_shape=jax.ShapeDtypeStruct((N, NCLS), jnp.float32),
    )(h, W, b)


def kernel(x, edge_index, W0, b0, Wconvs, Wout, bout):
    vertex2d = edge_index[0].reshape(ROWS, K)
    edges2d = edge_index[1].reshape(ROWS, K)
    vA = vertex2d * 2
    vB = vA + 1
    eA = edges2d * 2
    eB = eA + 1

    invv, inve = _sc_counts(vertex2d, edges2d)

    h = _tc_linear_relu(x, W0, b0.reshape(1, D))
    h0 = h
    invv1 = invv[:N]
    inve1 = inve[:N]
    for i in range(NLAYERS):
        beta = math.log(LAMDA / (i + 1) + 1.0)
        pe = _sc_gather_scatter(h.reshape(NC * N, H), vA, vB, edges2d)[0]
        xe = _tc_combine(pe, inve1)
        pv = _sc_gather_scatter(xe.reshape(NC * N, H), eA, eB, vertex2d)[0]
        h = _tc_layer(pv, invv1, h0, Wconvs[i], beta)

    return _tc_out(h, Wout, bout.reshape(1, NCLS))
